# baseline (speedup 1.0000x reference)
import jax
import jax.numpy as jnp
from jax.experimental import pallas as pl

_ROW_TILE = 5008


def _body(x_ref, pw_ref, out_ref, h_ref):
    hid = h_ref.shape[1]
    out_dim = out_ref.shape[1]
    wcat = pw_ref[0:128, :]
    bcat = pw_ref[128:129, :]
    wl = pw_ref[136:168, 0:out_dim]
    bl = pw_ref[168:169, 0:out_dim]
    xb = x_ref[...]
    s = jax.nn.sigmoid(
        jnp.dot(xb, wcat, preferred_element_type=jnp.float32) + bcat)
    z = s[:, :hid]
    h_tilde = 2.0 * s[:, hid:] - 1.0
    h = (1.0 - z) * h_tilde
    h_ref[...] = h
    out_ref[...] = (
        jnp.dot(jnp.maximum(h, 0.0), wl,
                preferred_element_type=jnp.float32) + bl)


def kernel(x, edge_index, edge_weight, W_z, b_z, W_r, b_r, W_h, b_h,
           W_lin, b_lin):
    n, in_dim = x.shape
    hid = W_z.shape[-1]
    out_dim = W_lin.shape[-1]

    wz = W_z[0, 0, :in_dim, :] + W_z[1, 0, :in_dim, :]
    wh = 2.0 * (W_h[0, 0, :in_dim, :] + W_h[1, 0, :in_dim, :])
    wcat = jnp.concatenate([wz, wh], axis=1)
    bcat = jnp.concatenate([b_z, 2.0 * b_h]).reshape(1, 2 * hid)
    wl64 = jnp.pad(W_lin, ((0, 0), (0, 2 * hid - out_dim)))
    bl64 = jnp.pad(b_lin.reshape(1, out_dim),
                   ((0, 0), (0, 2 * hid - out_dim)))
    pw = jnp.concatenate([
        wcat,
        jnp.pad(bcat, ((0, 7), (0, 0))),
        wl64,
        jnp.pad(bl64, ((0, 7), (0, 0))),
    ], axis=0)

    out, h = pl.pallas_call(
        _body,
        grid=(pl.cdiv(n, _ROW_TILE),),
        in_specs=[
            pl.BlockSpec((_ROW_TILE, in_dim), lambda i: (i, 0)),
            pl.BlockSpec(pw.shape, lambda i: (0, 0)),
        ],
        out_specs=[
            pl.BlockSpec((_ROW_TILE, out_dim), lambda i: (i, 0)),
            pl.BlockSpec((_ROW_TILE, hid), lambda i: (i, 0)),
        ],
        out_shape=[
            jax.ShapeDtypeStruct((n, out_dim), x.dtype),
            jax.ShapeDtypeStruct((n, hid), x.dtype),
        ],
    )(x, pw)
    return (out, h)
